# baseline (speedup 1.0000x reference)
import functools

import jax
import jax.numpy as jnp
from jax import lax
from jax.experimental import pallas as pl
from jax.experimental.pallas import tpu as pltpu
from jax.experimental.pallas import tpu_sc as plsc

H = 8
DH = 16
EPS = 1e-16
NEG_SLOPE = 0.2


def _dense1_body(x_ref, w_ref, asrc_ref, adst_ref, h_ref, ss_ref, sd_ref):
    x = x_ref[...]
    h = jnp.dot(x, w_ref[...], preferred_element_type=jnp.float32)
    h_ref[...] = h
    ss_ref[...] = jnp.dot(h, asrc_ref[...], preferred_element_type=jnp.float32)
    sd_ref[...] = jnp.dot(h, adst_ref[...], preferred_element_type=jnp.float32)


def _dense1(x, w, asrc, adst, bn):
    n, d = x.shape
    grid = n // bn
    return pl.pallas_call(
        _dense1_body,
        grid=(grid,),
        in_specs=[
            pl.BlockSpec((bn, d), lambda i: (i, 0)),
            pl.BlockSpec((d, d), lambda i: (0, 0)),
            pl.BlockSpec((d, DH), lambda i: (0, 0)),
            pl.BlockSpec((d, DH), lambda i: (0, 0)),
        ],
        out_specs=[
            pl.BlockSpec((bn, d), lambda i: (i, 0)),
            pl.BlockSpec((bn, DH), lambda i: (i, 0)),
            pl.BlockSpec((bn, DH), lambda i: (i, 0)),
        ],
        out_shape=[
            jax.ShapeDtypeStruct((n, d), jnp.float32),
            jax.ShapeDtypeStruct((n, DH), jnp.float32),
            jax.ShapeDtypeStruct((n, DH), jnp.float32),
        ],
    )(x, w, asrc, adst)


def _dense2_body(msg_ref, den_ref, p_ref, w_ref, asrc_ref, adst_ref,
                 h_ref, ss_ref, sd_ref):
    msg = msg_ref[0] + msg_ref[1]
    den = den_ref[0] + den_ref[1]
    den_exp = jnp.dot(den, p_ref[...], preferred_element_type=jnp.float32)
    out = msg / (den_exp + EPS)
    x = jnp.where(out > 0.0, out, jnp.exp(out) - 1.0)
    h = jnp.dot(x, w_ref[...], preferred_element_type=jnp.float32)
    h_ref[...] = h
    ss_ref[...] = jnp.dot(h, asrc_ref[...], preferred_element_type=jnp.float32)
    sd_ref[...] = jnp.dot(h, adst_ref[...], preferred_element_type=jnp.float32)


def _dense2(msg, den, p, w, asrc, adst, bn):
    _, n, d = msg.shape
    grid = n // bn
    return pl.pallas_call(
        _dense2_body,
        grid=(grid,),
        in_specs=[
            pl.BlockSpec((2, bn, d), lambda i: (0, i, 0)),
            pl.BlockSpec((2, bn, DH), lambda i: (0, i, 0)),
            pl.BlockSpec((DH, d), lambda i: (0, 0)),
            pl.BlockSpec((d, d), lambda i: (0, 0)),
            pl.BlockSpec((d, DH), lambda i: (0, 0)),
            pl.BlockSpec((d, DH), lambda i: (0, 0)),
        ],
        out_specs=[
            pl.BlockSpec((bn, d), lambda i: (i, 0)),
            pl.BlockSpec((bn, DH), lambda i: (i, 0)),
            pl.BlockSpec((bn, DH), lambda i: (i, 0)),
        ],
        out_shape=[
            jax.ShapeDtypeStruct((n, d), jnp.float32),
            jax.ShapeDtypeStruct((n, DH), jnp.float32),
            jax.ShapeDtypeStruct((n, DH), jnp.float32),
        ],
    )(msg, den, p, w, asrc, adst)


def _final_body(msg_ref, den_ref, p_ref, out_ref):
    msg = msg_ref[0] + msg_ref[1]
    den = den_ref[0] + den_ref[1]
    den_exp = jnp.dot(den, p_ref[...], preferred_element_type=jnp.float32)
    out_ref[...] = msg / (den_exp + EPS)


def _final(msg, den, p, bn):
    _, n, d = msg.shape
    grid = n // bn
    return pl.pallas_call(
        _final_body,
        grid=(grid,),
        in_specs=[
            pl.BlockSpec((2, bn, d), lambda i: (0, i, 0)),
            pl.BlockSpec((2, bn, DH), lambda i: (0, i, 0)),
            pl.BlockSpec((DH, d), lambda i: (0, 0)),
        ],
        out_specs=pl.BlockSpec((bn, d), lambda i: (i, 0)),
        out_shape=jax.ShapeDtypeStruct((n, d), jnp.float32),
    )(msg, den, p)


def _temporal_body(te_ref, ta_ref, w1_ref, b1_ref, w2_ref, b2_ref,
                   w3_ref, b3_ref, out_ref):
    a = ta_ref[...]
    t1 = jnp.dot(te_ref[...], w1_ref[...], preferred_element_type=jnp.float32)
    t1 = jnp.dot(a, t1, preferred_element_type=jnp.float32) + b1_ref[...]
    t1 = jnp.maximum(t1, 0.0)
    t2 = jnp.dot(t1, w2_ref[...], preferred_element_type=jnp.float32)
    t2 = jnp.dot(a, t2, preferred_element_type=jnp.float32) + b2_ref[...]
    t2 = jnp.maximum(t2, 0.0)
    t3 = jnp.dot(t2, w3_ref[...], preferred_element_type=jnp.float32)
    out_ref[...] = jnp.dot(a, t3, preferred_element_type=jnp.float32) + b3_ref[...]


def _temporal(t_emb, t_adj, w1, b1, w2, b2, w3, b3):
    t, d = t_emb.shape
    return pl.pallas_call(
        _temporal_body,
        out_shape=jax.ShapeDtypeStruct((t, d), jnp.float32),
    )(t_emb, t_adj, w1, b1.reshape(1, -1), w2, b2.reshape(1, -1),
      w3, b3.reshape(1, -1))


CH = 64

_GD = lax.GatherDimensionNumbers(
    offset_dims=(), collapsed_slice_dims=(0,), start_index_map=(0,))


def _lane_splat(v, lane):
    idx = jnp.full((DH, 1), lane, jnp.int32)
    return lax.gather(v, idx, _GD, (1,),
                      mode=lax.GatherScatterMode.PROMISE_IN_BOUNDS)


def _make_edge_kernel(n, cpt, d):
    info = plsc.get_sparse_core_info()
    nc, ns = info.num_cores, info.num_subcores
    nw = nc * ns
    rpt = n // ns
    assert rpt * ns == n and cpt % 2 == 0
    mesh = plsc.VectorSubcoreMesh(core_axis_name="c", subcore_axis_name="s")

    slot = lambda: (
        pltpu.VMEM((CH,), jnp.int32),
        pltpu.VMEM((CH,), jnp.int32),
        pltpu.VMEM((CH,), jnp.int32),
        pltpu.VMEM((CH, DH), jnp.float32),
        pltpu.VMEM((CH, DH), jnp.float32),
        pltpu.VMEM((CH, d), jnp.float32),
        pltpu.VMEM((CH, DH), jnp.float32),
        pltpu.VMEM((CH, d), jnp.float32),
        pltpu.SemaphoreType.DMA,
        pltpu.SemaphoreType.DMA,
    )

    @functools.partial(
        pl.kernel,
        mesh=mesh,
        compiler_params=pltpu.CompilerParams(use_tc_tiling_on_sc=False),
        out_type=(
            jax.ShapeDtypeStruct((nc, n, d), jnp.float32),
            jax.ShapeDtypeStruct((nc, n, DH), jnp.float32),
        ),
        scratch_types=(
            pltpu.VMEM_SHARED((n, d), jnp.float32),
            pltpu.VMEM_SHARED((n, DH), jnp.float32),
        ) + slot() + slot(),
    )
    def edge_kernel(h_hbm, ssrc_hbm, sdst_hbm, src_hbm, dst_hbm,
                    zmsg_hbm, zden_hbm, msg_out, den_out,
                    msg_acc, den_acc, *slots):
        c = lax.axis_index("c")
        s = lax.axis_index("s")
        tid = s * nc + c
        sl = [slots[:10], slots[10:]]

        pltpu.sync_copy(zmsg_hbm, msg_acc.at[pl.ds(s * rpt, rpt)])
        pltpu.sync_copy(zden_hbm, den_acc.at[pl.ds(s * rpt, rpt)])
        plsc.subcore_barrier()

        def load_idx_and_gather(b, i):
            srcv, dstv, _, ssv, sdv, hv, _, _, gsem, _ = sl[b]
            base = (tid * cpt + i) * CH
            pltpu.sync_copy(src_hbm.at[pl.ds(base, CH)], srcv)
            pltpu.sync_copy(dst_hbm.at[pl.ds(base, CH)], dstv)
            pltpu.async_copy(ssrc_hbm.at[srcv], ssv, gsem)
            pltpu.async_copy(sdst_hbm.at[dstv], sdv, gsem)
            pltpu.async_copy(h_hbm.at[srcv], hv, gsem)

        def wait_gathers(b):
            srcv, dstv, _, ssv, sdv, hv, _, _, gsem, _ = sl[b]
            pltpu.make_async_copy(ssrc_hbm.at[srcv], ssv, gsem).wait()
            pltpu.make_async_copy(sdst_hbm.at[dstv], sdv, gsem).wait()
            pltpu.make_async_copy(h_hbm.at[srcv], hv, gsem).wait()

        def wait_scatters(b):
            _, _, dstsv, _, _, _, wv, msgv, _, ssem = sl[b]
            pltpu.make_async_copy(wv, den_acc.at[dstsv], ssem).wait()
            pltpu.make_async_copy(msgv, msg_acc.at[dstsv], ssem).wait()

        def compute(b):
            _, _, _, ssv, sdv, hv, wv, msgv, _, _ = sl[b]

            def edge_body(j, carry2):
                sc = ssv[j] + sdv[j]
                ew = jnp.exp(jnp.maximum(sc, NEG_SLOPE * sc))
                wv[j] = ew
                for hh in range(H):
                    sp = _lane_splat(ew, hh)
                    msgv[j, pl.ds(hh * DH, DH)] = hv[j, pl.ds(hh * DH, DH)] * sp
                return carry2

            lax.fori_loop(0, CH, edge_body, 0)

        def issue_scatters(b):
            _, dstv, dstsv, _, _, _, wv, msgv, _, ssem = sl[b]
            for jj in range(CH // DH):
                dstsv[pl.ds(jj * DH, DH)] = dstv[pl.ds(jj * DH, DH)]
            pltpu.async_copy(wv, den_acc.at[dstsv], ssem, add=True)
            pltpu.async_copy(msgv, msg_acc.at[dstsv], ssem, add=True)

        for b in (0, 1):
            load_idx_and_gather(b, i=jnp.int32(b))

        def pair_body(p, carry):
            for b in (0, 1):
                i = 2 * p + b
                wait_gathers(b)
                pl.when(i >= 2)(lambda: wait_scatters(b))
                compute(b)
                issue_scatters(b)
                load_idx_and_gather(b, i + 2)
            return carry

        lax.fori_loop(0, cpt // 2, pair_body, 0)

        for b in (0, 1):
            wait_gathers(b)
            wait_scatters(b)

        plsc.subcore_barrier()
        pltpu.sync_copy(msg_acc.at[pl.ds(s * rpt, rpt)],
                        msg_out.at[c, pl.ds(s * rpt, rpt)])
        pltpu.sync_copy(den_acc.at[pl.ds(s * rpt, rpt)],
                        den_out.at[c, pl.ds(s * rpt, rpt)])

    return edge_kernel


def _score_mat(a):
    h, dh = a.shape
    d = h * dh
    rows = jnp.arange(d) // dh
    cols = jnp.arange(DH)
    return jnp.where(rows[:, None] == cols[None, :],
                     a.reshape(-1)[:, None], 0.0).astype(jnp.float32)


def kernel(sp_x, edge_index, t_emb, t_adj, Wg0, a_src0, a_dst0,
           Wg1, a_src1, a_dst1, W1, b1, W2, b2, W3, b3):
    n, d = sp_x.shape
    e = edge_index.shape[1]
    src = edge_index[0]
    dst = edge_index[1]

    asrc0 = _score_mat(a_src0)
    adst0 = _score_mat(a_dst0)
    asrc1 = _score_mat(a_src1)
    adst1 = _score_mat(a_dst1)
    jj = jnp.arange(d) // DH
    p = (jnp.arange(DH)[:, None] == jj[None, :]).astype(jnp.float32)

    info = plsc.get_sparse_core_info()
    ns = info.num_subcores
    np2 = ((n + 8 * ns - 1) // (8 * ns)) * (8 * ns)
    rpt = np2 // ns
    zmsg = jnp.zeros((rpt, d), jnp.float32)
    zden = jnp.zeros((rpt, DH), jnp.float32)

    nw = info.num_cores * ns
    cpt = -(-e // (CH * nw))
    cpt += cpt % 2
    e3 = (cpt * nw + 2) * CH
    pad_idx = jnp.full((e3 - e,), n, jnp.int32)
    src = jnp.concatenate([src, pad_idx])
    dst = jnp.concatenate([dst, pad_idx])

    edge_k = _make_edge_kernel(np2, cpt, d)

    bn = np2 // 16
    x0 = jnp.pad(sp_x, ((0, np2 - n), (0, 0)))
    h1, ss1, sd1 = _dense1(x0, Wg0, asrc0, adst0, bn)
    msg1, den1 = edge_k(h1, ss1, sd1, src, dst, zmsg, zden)
    h2, ss2, sd2 = _dense2(msg1, den1, p, Wg1, asrc1, adst1, bn)
    msg2, den2 = edge_k(h2, ss2, sd2, src, dst, zmsg, zden)
    sp = _final(msg2, den2, p, bn)[:n]

    tp = _temporal(t_emb, t_adj, W1, b1, W2, b2, W3, b3)
    return (sp, tp)


# speedup vs baseline: 1.7371x; 1.7371x over previous
import functools

import jax
import jax.numpy as jnp
from jax import lax
from jax.experimental import pallas as pl
from jax.experimental.pallas import tpu as pltpu
from jax.experimental.pallas import tpu_sc as plsc

H = 8
DH = 16
EPS = 1e-16
NEG_SLOPE = 0.2


def _dense1_body(x_ref, w_ref, asrc_ref, adst_ref, h_ref, ss_ref, sd_ref):
    x = x_ref[...]
    h = jnp.dot(x, w_ref[...], preferred_element_type=jnp.float32)
    h_ref[...] = h
    ss_ref[...] = jnp.dot(h, asrc_ref[...], preferred_element_type=jnp.float32)
    sd_ref[...] = jnp.dot(h, adst_ref[...], preferred_element_type=jnp.float32)


def _dense1(x, w, asrc, adst, bn):
    n, d = x.shape
    grid = n // bn
    return pl.pallas_call(
        _dense1_body,
        grid=(grid,),
        in_specs=[
            pl.BlockSpec((bn, d), lambda i: (i, 0)),
            pl.BlockSpec((d, d), lambda i: (0, 0)),
            pl.BlockSpec((d, DH), lambda i: (0, 0)),
            pl.BlockSpec((d, DH), lambda i: (0, 0)),
        ],
        out_specs=[
            pl.BlockSpec((bn, d), lambda i: (i, 0)),
            pl.BlockSpec((bn, DH), lambda i: (i, 0)),
            pl.BlockSpec((bn, DH), lambda i: (i, 0)),
        ],
        out_shape=[
            jax.ShapeDtypeStruct((n, d), jnp.float32),
            jax.ShapeDtypeStruct((n, DH), jnp.float32),
            jax.ShapeDtypeStruct((n, DH), jnp.float32),
        ],
    )(x, w, asrc, adst)


def _dense2_body(msg_ref, den_ref, p_ref, w_ref, asrc_ref, adst_ref,
                 h_ref, ss_ref, sd_ref):
    msg = msg_ref[0] + msg_ref[1]
    den = den_ref[0] + den_ref[1]
    den_exp = jnp.dot(den, p_ref[...], preferred_element_type=jnp.float32)
    out = msg / (den_exp + EPS)
    x = jnp.where(out > 0.0, out, jnp.exp(out) - 1.0)
    h = jnp.dot(x, w_ref[...], preferred_element_type=jnp.float32)
    h_ref[...] = h
    ss_ref[...] = jnp.dot(h, asrc_ref[...], preferred_element_type=jnp.float32)
    sd_ref[...] = jnp.dot(h, adst_ref[...], preferred_element_type=jnp.float32)


def _dense2(msg, den, p, w, asrc, adst, bn):
    _, n, d = msg.shape
    grid = n // bn
    return pl.pallas_call(
        _dense2_body,
        grid=(grid,),
        in_specs=[
            pl.BlockSpec((2, bn, d), lambda i: (0, i, 0)),
            pl.BlockSpec((2, bn, DH), lambda i: (0, i, 0)),
            pl.BlockSpec((DH, d), lambda i: (0, 0)),
            pl.BlockSpec((d, d), lambda i: (0, 0)),
            pl.BlockSpec((d, DH), lambda i: (0, 0)),
            pl.BlockSpec((d, DH), lambda i: (0, 0)),
        ],
        out_specs=[
            pl.BlockSpec((bn, d), lambda i: (i, 0)),
            pl.BlockSpec((bn, DH), lambda i: (i, 0)),
            pl.BlockSpec((bn, DH), lambda i: (i, 0)),
        ],
        out_shape=[
            jax.ShapeDtypeStruct((n, d), jnp.float32),
            jax.ShapeDtypeStruct((n, DH), jnp.float32),
            jax.ShapeDtypeStruct((n, DH), jnp.float32),
        ],
    )(msg, den, p, w, asrc, adst)


def _final_body(msg_ref, den_ref, p_ref, out_ref):
    msg = msg_ref[0] + msg_ref[1]
    den = den_ref[0] + den_ref[1]
    den_exp = jnp.dot(den, p_ref[...], preferred_element_type=jnp.float32)
    out_ref[...] = msg / (den_exp + EPS)


def _final(msg, den, p, bn):
    _, n, d = msg.shape
    grid = n // bn
    return pl.pallas_call(
        _final_body,
        grid=(grid,),
        in_specs=[
            pl.BlockSpec((2, bn, d), lambda i: (0, i, 0)),
            pl.BlockSpec((2, bn, DH), lambda i: (0, i, 0)),
            pl.BlockSpec((DH, d), lambda i: (0, 0)),
        ],
        out_specs=pl.BlockSpec((bn, d), lambda i: (i, 0)),
        out_shape=jax.ShapeDtypeStruct((n, d), jnp.float32),
    )(msg, den, p)


def _temporal_body(te_ref, ta_ref, w1_ref, b1_ref, w2_ref, b2_ref,
                   w3_ref, b3_ref, out_ref):
    a = ta_ref[...]
    t1 = jnp.dot(te_ref[...], w1_ref[...], preferred_element_type=jnp.float32)
    t1 = jnp.dot(a, t1, preferred_element_type=jnp.float32) + b1_ref[...]
    t1 = jnp.maximum(t1, 0.0)
    t2 = jnp.dot(t1, w2_ref[...], preferred_element_type=jnp.float32)
    t2 = jnp.dot(a, t2, preferred_element_type=jnp.float32) + b2_ref[...]
    t2 = jnp.maximum(t2, 0.0)
    t3 = jnp.dot(t2, w3_ref[...], preferred_element_type=jnp.float32)
    out_ref[...] = jnp.dot(a, t3, preferred_element_type=jnp.float32) + b3_ref[...]


def _temporal(t_emb, t_adj, w1, b1, w2, b2, w3, b3):
    t, d = t_emb.shape
    return pl.pallas_call(
        _temporal_body,
        out_shape=jax.ShapeDtypeStruct((t, d), jnp.float32),
    )(t_emb, t_adj, w1, b1.reshape(1, -1), w2, b2.reshape(1, -1),
      w3, b3.reshape(1, -1))


CH = 64

_GD = lax.GatherDimensionNumbers(
    offset_dims=(), collapsed_slice_dims=(0,), start_index_map=(0,))


def _lane_splat(v, lane):
    idx = jnp.full((DH, 1), lane, jnp.int32)
    return lax.gather(v, idx, _GD, (1,),
                      mode=lax.GatherScatterMode.PROMISE_IN_BOUNDS)


def _make_edge_kernel(n, cpt, d):
    info = plsc.get_sparse_core_info()
    nc, ns = info.num_cores, info.num_subcores
    nw = nc * ns
    rpt = n // ns
    assert rpt * ns == n and cpt % 2 == 0
    mesh = plsc.VectorSubcoreMesh(core_axis_name="c", subcore_axis_name="s")

    slot = lambda: (
        pltpu.VMEM((CH,), jnp.int32),
        pltpu.VMEM((CH,), jnp.int32),
        pltpu.VMEM((CH,), jnp.int32),
        pltpu.VMEM((CH, DH), jnp.float32),
        pltpu.VMEM((CH, DH), jnp.float32),
        pltpu.VMEM((CH, d), jnp.float32),
        pltpu.VMEM((CH, DH), jnp.float32),
        pltpu.VMEM((CH, d), jnp.float32),
        pltpu.SemaphoreType.DMA,
        pltpu.SemaphoreType.DMA,
    )

    @functools.partial(
        pl.kernel,
        mesh=mesh,
        compiler_params=pltpu.CompilerParams(use_tc_tiling_on_sc=False),
        out_type=(
            jax.ShapeDtypeStruct((nc, n, d), jnp.float32),
            jax.ShapeDtypeStruct((nc, n, DH), jnp.float32),
        ),
        scratch_types=(
            pltpu.VMEM_SHARED((n, d), jnp.float32),
            pltpu.VMEM_SHARED((n, DH), jnp.float32),
        ) + slot() + slot(),
    )
    def edge_kernel(h_hbm, ssrc_hbm, sdst_hbm, src_hbm, dst_hbm,
                    zmsg_hbm, zden_hbm, msg_out, den_out,
                    msg_acc, den_acc, *slots):
        c = lax.axis_index("c")
        s = lax.axis_index("s")
        tid = s * nc + c
        sl = [slots[:10], slots[10:]]

        pltpu.sync_copy(zmsg_hbm, msg_acc.at[pl.ds(s * rpt, rpt)])
        pltpu.sync_copy(zden_hbm, den_acc.at[pl.ds(s * rpt, rpt)])
        plsc.subcore_barrier()

        def load_idx_and_gather(b, i):
            srcv, dstv, _, ssv, sdv, hv, _, _, gsem, _ = sl[b]
            base = (tid * cpt + i) * CH
            pltpu.sync_copy(src_hbm.at[pl.ds(base, CH)], srcv)
            pltpu.sync_copy(dst_hbm.at[pl.ds(base, CH)], dstv)
            pltpu.async_copy(ssrc_hbm.at[srcv], ssv, gsem)
            pltpu.async_copy(sdst_hbm.at[dstv], sdv, gsem)
            pltpu.async_copy(h_hbm.at[srcv], hv, gsem)

        def wait_gathers(b):
            srcv, dstv, _, ssv, sdv, hv, _, _, gsem, _ = sl[b]
            pltpu.make_async_copy(ssrc_hbm.at[srcv], ssv, gsem).wait()
            pltpu.make_async_copy(sdst_hbm.at[dstv], sdv, gsem).wait()
            pltpu.make_async_copy(h_hbm.at[srcv], hv, gsem).wait()

        def wait_scatters(b):
            _, _, dstsv, _, _, _, wv, msgv, _, ssem = sl[b]
            pltpu.make_async_copy(wv, den_acc.at[dstsv], ssem).wait()
            pltpu.make_async_copy(msgv, msg_acc.at[dstsv], ssem).wait()

        def compute(b):
            _, _, _, ssv, sdv, hv, wv, msgv, _, _ = sl[b]

            @plsc.parallel_loop(0, CH, 1, unroll=4)
            def _(j):
                sc = ssv[j] + sdv[j]
                ew = jnp.exp(jnp.maximum(sc, NEG_SLOPE * sc))
                wv[j] = ew
                for hh in range(H):
                    sp = _lane_splat(ew, hh)
                    msgv[j, pl.ds(hh * DH, DH)] = hv[j, pl.ds(hh * DH, DH)] * sp

        def issue_scatters(b):
            _, dstv, dstsv, _, _, _, wv, msgv, _, ssem = sl[b]
            for jj in range(CH // DH):
                dstsv[pl.ds(jj * DH, DH)] = dstv[pl.ds(jj * DH, DH)]
            pltpu.async_copy(wv, den_acc.at[dstsv], ssem, add=True)
            pltpu.async_copy(msgv, msg_acc.at[dstsv], ssem, add=True)

        for b in (0, 1):
            load_idx_and_gather(b, i=jnp.int32(b))

        def pair_body(p, carry):
            for b in (0, 1):
                i = 2 * p + b
                wait_gathers(b)
                pl.when(i >= 2)(lambda: wait_scatters(b))
                compute(b)
                issue_scatters(b)
                load_idx_and_gather(b, i + 2)
            return carry

        lax.fori_loop(0, cpt // 2, pair_body, 0)

        for b in (0, 1):
            wait_gathers(b)
            wait_scatters(b)

        plsc.subcore_barrier()
        pltpu.sync_copy(msg_acc.at[pl.ds(s * rpt, rpt)],
                        msg_out.at[c, pl.ds(s * rpt, rpt)])
        pltpu.sync_copy(den_acc.at[pl.ds(s * rpt, rpt)],
                        den_out.at[c, pl.ds(s * rpt, rpt)])

    return edge_kernel


def _score_mat(a):
    h, dh = a.shape
    d = h * dh
    rows = jnp.arange(d) // dh
    cols = jnp.arange(DH)
    return jnp.where(rows[:, None] == cols[None, :],
                     a.reshape(-1)[:, None], 0.0).astype(jnp.float32)


def kernel(sp_x, edge_index, t_emb, t_adj, Wg0, a_src0, a_dst0,
           Wg1, a_src1, a_dst1, W1, b1, W2, b2, W3, b3):
    n, d = sp_x.shape
    e = edge_index.shape[1]
    src = edge_index[0]
    dst = edge_index[1]

    asrc0 = _score_mat(a_src0)
    adst0 = _score_mat(a_dst0)
    asrc1 = _score_mat(a_src1)
    adst1 = _score_mat(a_dst1)
    jj = jnp.arange(d) // DH
    p = (jnp.arange(DH)[:, None] == jj[None, :]).astype(jnp.float32)

    info = plsc.get_sparse_core_info()
    ns = info.num_subcores
    np2 = ((n + 8 * ns - 1) // (8 * ns)) * (8 * ns)
    rpt = np2 // ns
    zmsg = jnp.zeros((rpt, d), jnp.float32)
    zden = jnp.zeros((rpt, DH), jnp.float32)

    nw = info.num_cores * ns
    cpt = -(-e // (CH * nw))
    cpt += cpt % 2
    e3 = (cpt * nw + 2) * CH
    pad_idx = jnp.full((e3 - e,), n, jnp.int32)
    src = jnp.concatenate([src, pad_idx])
    dst = jnp.concatenate([dst, pad_idx])

    edge_k = _make_edge_kernel(np2, cpt, d)

    bn = np2 // 16
    x0 = jnp.pad(sp_x, ((0, np2 - n), (0, 0)))
    h1, ss1, sd1 = _dense1(x0, Wg0, asrc0, adst0, bn)
    msg1, den1 = edge_k(h1, ss1, sd1, src, dst, zmsg, zden)
    h2, ss2, sd2 = _dense2(msg1, den1, p, Wg1, asrc1, adst1, bn)
    msg2, den2 = edge_k(h2, ss2, sd2, src, dst, zmsg, zden)
    sp = _final(msg2, den2, p, bn)[:n]

    tp = _temporal(t_emb, t_adj, W1, b1, W2, b2, W3, b3)
    return (sp, tp)


# speedup vs baseline: 2.0084x; 1.1562x over previous
import functools

import jax
import jax.numpy as jnp
from jax import lax
from jax.experimental import pallas as pl
from jax.experimental.pallas import tpu as pltpu
from jax.experimental.pallas import tpu_sc as plsc

H = 8
DH = 16
D = 128
DW = D + DH
EPS = 1e-16
NEG_SLOPE = 0.2


def _dense1_body(x_ref, w_ref, wd_ref, hs_ref, sd_ref):
    x = x_ref[...]
    hs_ref[...] = jnp.dot(x, w_ref[...], preferred_element_type=jnp.float32)
    sd_ref[...] = jnp.dot(x, wd_ref[...], preferred_element_type=jnp.float32)


def _dense1(x, w144, wd, bn):
    n, d = x.shape
    return pl.pallas_call(
        _dense1_body,
        grid=(n // bn,),
        in_specs=[
            pl.BlockSpec((bn, d), lambda i: (i, 0)),
            pl.BlockSpec((d, DW), lambda i: (0, 0)),
            pl.BlockSpec((d, DH), lambda i: (0, 0)),
        ],
        out_specs=[
            pl.BlockSpec((bn, DW), lambda i: (i, 0)),
            pl.BlockSpec((bn, DH), lambda i: (i, 0)),
        ],
        out_shape=[
            jax.ShapeDtypeStruct((n, DW), jnp.float32),
            jax.ShapeDtypeStruct((n, DH), jnp.float32),
        ],
    )(x, w144, wd)


def _dense2_body(msg_ref, p2_ref, w_ref, wd_ref, hs_ref, sd_ref):
    m = msg_ref[0] + msg_ref[1]
    den = jnp.dot(m, p2_ref[...], preferred_element_type=jnp.float32)
    out = m[:, :D] / (den + EPS)
    x = jnp.where(out > 0.0, out, jnp.exp(out) - 1.0)
    hs_ref[...] = jnp.dot(x, w_ref[...], preferred_element_type=jnp.float32)
    sd_ref[...] = jnp.dot(x, wd_ref[...], preferred_element_type=jnp.float32)


def _dense2(msg, p2, w144, wd, bn):
    _, n, _ = msg.shape
    return pl.pallas_call(
        _dense2_body,
        grid=(n // bn,),
        in_specs=[
            pl.BlockSpec((2, bn, DW), lambda i: (0, i, 0)),
            pl.BlockSpec((DW, D), lambda i: (0, 0)),
            pl.BlockSpec((D, DW), lambda i: (0, 0)),
            pl.BlockSpec((D, DH), lambda i: (0, 0)),
        ],
        out_specs=[
            pl.BlockSpec((bn, DW), lambda i: (i, 0)),
            pl.BlockSpec((bn, DH), lambda i: (i, 0)),
        ],
        out_shape=[
            jax.ShapeDtypeStruct((n, DW), jnp.float32),
            jax.ShapeDtypeStruct((n, DH), jnp.float32),
        ],
    )(msg, p2, w144, wd)


def _final_body(msg_ref, p2_ref, out_ref):
    m = msg_ref[0] + msg_ref[1]
    den = jnp.dot(m, p2_ref[...], preferred_element_type=jnp.float32)
    out_ref[...] = m[:, :D] / (den + EPS)


def _final(msg, p2, bn):
    _, n, _ = msg.shape
    return pl.pallas_call(
        _final_body,
        grid=(n // bn,),
        in_specs=[
            pl.BlockSpec((2, bn, DW), lambda i: (0, i, 0)),
            pl.BlockSpec((DW, D), lambda i: (0, 0)),
        ],
        out_specs=pl.BlockSpec((bn, D), lambda i: (i, 0)),
        out_shape=jax.ShapeDtypeStruct((n, D), jnp.float32),
    )(msg, p2)


def _temporal_body(te_ref, ta_ref, w1_ref, b1_ref, w2_ref, b2_ref,
                   w3_ref, b3_ref, out_ref):
    a = ta_ref[...]
    t1 = jnp.dot(te_ref[...], w1_ref[...], preferred_element_type=jnp.float32)
    t1 = jnp.dot(a, t1, preferred_element_type=jnp.float32) + b1_ref[...]
    t1 = jnp.maximum(t1, 0.0)
    t2 = jnp.dot(t1, w2_ref[...], preferred_element_type=jnp.float32)
    t2 = jnp.dot(a, t2, preferred_element_type=jnp.float32) + b2_ref[...]
    t2 = jnp.maximum(t2, 0.0)
    t3 = jnp.dot(t2, w3_ref[...], preferred_element_type=jnp.float32)
    out_ref[...] = jnp.dot(a, t3, preferred_element_type=jnp.float32) + b3_ref[...]


def _temporal(t_emb, t_adj, w1, b1, w2, b2, w3, b3):
    t, d = t_emb.shape
    return pl.pallas_call(
        _temporal_body,
        out_shape=jax.ShapeDtypeStruct((t, d), jnp.float32),
    )(t_emb, t_adj, w1, b1.reshape(1, -1), w2, b2.reshape(1, -1),
      w3, b3.reshape(1, -1))


CH = 80
BLK = 6

_GD = lax.GatherDimensionNumbers(
    offset_dims=(), collapsed_slice_dims=(0,), start_index_map=(0,))


def _lane_splat(v, lane):
    idx = jnp.full((DH, 1), lane, jnp.int32)
    return lax.gather(v, idx, _GD, (1,),
                      mode=lax.GatherScatterMode.PROMISE_IN_BOUNDS)


def _make_edge_kernel(n, cpt):
    info = plsc.get_sparse_core_info()
    nc, ns = info.num_cores, info.num_subcores
    rpt = n // ns
    assert rpt * ns == n and cpt % BLK == 0
    nblk = cpt // BLK
    mesh = plsc.VectorSubcoreMesh(core_axis_name="c", subcore_axis_name="s")

    slot = lambda: (
        pltpu.VMEM((CH, DW), jnp.float32),
        pltpu.VMEM((CH, DH), jnp.float32),
        pltpu.VMEM((CH,), jnp.int32),
        pltpu.SemaphoreType.DMA,
        pltpu.SemaphoreType.DMA,
    )

    @functools.partial(
        pl.kernel,
        mesh=mesh,
        compiler_params=pltpu.CompilerParams(use_tc_tiling_on_sc=False),
        out_type=jax.ShapeDtypeStruct((nc, n, DW), jnp.float32),
        scratch_types=(
            pltpu.VMEM_SHARED((n, DW), jnp.float32),
            pltpu.VMEM((BLK * CH,), jnp.int32),
            pltpu.VMEM((BLK * CH,), jnp.int32),
        ) + slot() + slot() + slot(),
    )
    def edge_kernel(hs_hbm, sd_hbm, src_hbm, dst_hbm, zmsg_hbm, msg_out,
                    acc, srcb, dstb, *slots):
        c = lax.axis_index("c")
        s = lax.axis_index("s")
        tid = s * nc + c
        sl = [slots[:5], slots[5:10], slots[10:]]

        pltpu.sync_copy(zmsg_hbm, acc.at[pl.ds(s * rpt, rpt)])
        plsc.subcore_barrier()

        def issue_gather(b, k):
            hsv, sdv, dstsv, gsem, _ = sl[b]
            off = k * CH
            for jj in range(CH // DH):
                dstsv[pl.ds(jj * DH, DH)] = dstb[pl.ds(off + jj * DH, DH)]
            pltpu.async_copy(hs_hbm.at[srcb.at[pl.ds(off, CH)]], hsv, gsem)
            pltpu.async_copy(sd_hbm.at[dstsv], sdv, gsem)

        def wait_gather(b):
            hsv, sdv, dstsv, gsem, _ = sl[b]
            pltpu.make_async_copy(hs_hbm.at[srcb.at[pl.ds(0, CH)]],
                                  hsv, gsem).wait()
            pltpu.make_async_copy(sd_hbm.at[dstsv], sdv, gsem).wait()

        def issue_scatter(b):
            hsv, _, dstsv, _, ssem = sl[b]
            pltpu.async_copy(hsv, acc.at[dstsv], ssem, add=True)

        def wait_scatter(b):
            hsv, _, dstsv, _, ssem = sl[b]
            pltpu.make_async_copy(hsv, acc.at[dstsv], ssem).wait()

        def compute(b):
            hsv, sdv, _, _, _ = sl[b]

            @plsc.parallel_loop(0, CH, 1, unroll=4)
            def _(j):
                sc = hsv[j, pl.ds(D, DH)] + sdv[j]
                ew = jnp.exp(jnp.maximum(sc, NEG_SLOPE * sc))
                hsv[j, pl.ds(D, DH)] = ew
                for hh in range(H):
                    sp = _lane_splat(ew, hh)
                    hsv[j, pl.ds(hh * DH, DH)] = hsv[j, pl.ds(hh * DH, DH)] * sp

        def process(b):
            wait_gather(b)
            compute(b)
            issue_scatter(b)

        def blk_body(g, carry):
            base = (tid * cpt + g * BLK) * CH
            pltpu.sync_copy(src_hbm.at[pl.ds(base, BLK * CH)], srcb)
            pltpu.sync_copy(dst_hbm.at[pl.ds(base, BLK * CH)], dstb)
            for k in range(BLK):
                s_cur = k % 3
                s_prc = (k - 1) % 3
                if k >= 3:
                    wait_scatter(s_cur)
                else:
                    pl.when(g > 0)(lambda b=s_cur: wait_scatter(b))
                issue_gather(s_cur, k)
                if k >= 1:
                    process(s_prc)
                else:
                    pl.when(g > 0)(lambda b=s_prc: process(b))
            return carry

        lax.fori_loop(0, nblk, blk_body, 0)

        process((cpt - 1) % 3)
        for b in range(3):
            wait_scatter(b)

        plsc.subcore_barrier()
        pltpu.sync_copy(acc.at[pl.ds(s * rpt, rpt)],
                        msg_out.at[c, pl.ds(s * rpt, rpt)])

    return edge_kernel


def _score_mat(a):
    h, dh = a.shape
    d = h * dh
    rows = jnp.arange(d) // dh
    cols = jnp.arange(DH)
    return jnp.where(rows[:, None] == cols[None, :],
                     a.reshape(-1)[:, None], 0.0).astype(jnp.float32)


def kernel(sp_x, edge_index, t_emb, t_adj, Wg0, a_src0, a_dst0,
           Wg1, a_src1, a_dst1, W1, b1, W2, b2, W3, b3):
    n, d = sp_x.shape
    e = edge_index.shape[1]
    src = edge_index[0]
    dst = edge_index[1]

    w144_0 = jnp.concatenate([Wg0, Wg0 @ _score_mat(a_src0)], axis=1)
    wd_0 = Wg0 @ _score_mat(a_dst0)
    w144_1 = jnp.concatenate([Wg1, Wg1 @ _score_mat(a_src1)], axis=1)
    wd_1 = Wg1 @ _score_mat(a_dst1)
    r = jnp.arange(DW)
    col_head = jnp.arange(D) // DH
    p2 = ((r[:, None] - D) == col_head[None, :]).astype(jnp.float32)

    info = plsc.get_sparse_core_info()
    nc, ns = info.num_cores, info.num_subcores
    nw = nc * ns
    np2 = ((n + 8 * ns - 1) // (8 * ns)) * (8 * ns)
    rpt = np2 // ns
    zmsg = jnp.zeros((rpt, DW), jnp.float32)

    cpt = -(-e // (CH * nw))
    cpt += (-cpt) % BLK
    e2 = cpt * nw * CH
    pad_idx = jnp.full((e2 - e,), n, jnp.int32)
    src = jnp.concatenate([src, pad_idx])
    dst = jnp.concatenate([dst, pad_idx])

    edge_k = _make_edge_kernel(np2, cpt)

    bn = np2 // 16
    x0 = jnp.pad(sp_x, ((0, np2 - n), (0, 0)))
    hs1, sd1 = _dense1(x0, w144_0, wd_0, bn)
    msg1 = edge_k(hs1, sd1, src, dst, zmsg)
    hs2, sd2 = _dense2(msg1, p2, w144_1, wd_1, bn)
    msg2 = edge_k(hs2, sd2, src, dst, zmsg)
    sp = _final(msg2, p2, bn)[:n]

    tp = _temporal(t_emb, t_adj, W1, b1, W2, b2, W3, b3)
    return (sp, tp)


# speedup vs baseline: 3.0520x; 1.5196x over previous
import functools

import jax
import jax.numpy as jnp
from jax import lax
from jax.experimental import pallas as pl
from jax.experimental.pallas import tpu as pltpu
from jax.experimental.pallas import tpu_sc as plsc

H = 8
DH = 16
D = 128
DW = D + DH
EPS = 1e-16
NEG_SLOPE = 0.2


def _dense1_body(x_ref, w_ref, wd_ref, hs_ref, sd_ref):
    x = x_ref[...]
    hs_ref[...] = jnp.dot(x, w_ref[...], preferred_element_type=jnp.float32)
    sd_ref[...] = jnp.dot(x, wd_ref[...], preferred_element_type=jnp.float32)


def _dense1(x, w144, wd, bn):
    n, d = x.shape
    return pl.pallas_call(
        _dense1_body,
        grid=(n // bn,),
        in_specs=[
            pl.BlockSpec((bn, d), lambda i: (i, 0)),
            pl.BlockSpec((d, DW), lambda i: (0, 0)),
            pl.BlockSpec((d, DH), lambda i: (0, 0)),
        ],
        out_specs=[
            pl.BlockSpec((bn, DW), lambda i: (i, 0)),
            pl.BlockSpec((bn, DH), lambda i: (i, 0)),
        ],
        out_shape=[
            jax.ShapeDtypeStruct((n, DW), jnp.float32),
            jax.ShapeDtypeStruct((n, DH), jnp.float32),
        ],
    )(x, w144, wd)


def _dense2_body(msg_ref, p2_ref, w_ref, wd_ref, hs_ref, sd_ref):
    m = msg_ref[0] + msg_ref[1]
    den = jnp.dot(m, p2_ref[...], preferred_element_type=jnp.float32)
    out = m[:, :D] / (den + EPS)
    x = jnp.where(out > 0.0, out, jnp.exp(out) - 1.0)
    hs_ref[...] = jnp.dot(x, w_ref[...], preferred_element_type=jnp.float32)
    sd_ref[...] = jnp.dot(x, wd_ref[...], preferred_element_type=jnp.float32)


def _dense2(msg, p2, w144, wd, bn):
    _, n, _ = msg.shape
    return pl.pallas_call(
        _dense2_body,
        grid=(n // bn,),
        in_specs=[
            pl.BlockSpec((2, bn, DW), lambda i: (0, i, 0)),
            pl.BlockSpec((DW, D), lambda i: (0, 0)),
            pl.BlockSpec((D, DW), lambda i: (0, 0)),
            pl.BlockSpec((D, DH), lambda i: (0, 0)),
        ],
        out_specs=[
            pl.BlockSpec((bn, DW), lambda i: (i, 0)),
            pl.BlockSpec((bn, DH), lambda i: (i, 0)),
        ],
        out_shape=[
            jax.ShapeDtypeStruct((n, DW), jnp.float32),
            jax.ShapeDtypeStruct((n, DH), jnp.float32),
        ],
    )(msg, p2, w144, wd)


def _final_body(msg_ref, p2_ref, out_ref):
    m = msg_ref[0] + msg_ref[1]
    den = jnp.dot(m, p2_ref[...], preferred_element_type=jnp.float32)
    out_ref[...] = m[:, :D] / (den + EPS)


def _final(msg, p2, bn):
    _, n, _ = msg.shape
    return pl.pallas_call(
        _final_body,
        grid=(n // bn,),
        in_specs=[
            pl.BlockSpec((2, bn, DW), lambda i: (0, i, 0)),
            pl.BlockSpec((DW, D), lambda i: (0, 0)),
        ],
        out_specs=pl.BlockSpec((bn, D), lambda i: (i, 0)),
        out_shape=jax.ShapeDtypeStruct((n, D), jnp.float32),
    )(msg, p2)


def _temporal_body(te_ref, ta_ref, w1_ref, b1_ref, w2_ref, b2_ref,
                   w3_ref, b3_ref, out_ref):
    a = ta_ref[...]
    t1 = jnp.dot(te_ref[...], w1_ref[...], preferred_element_type=jnp.float32)
    t1 = jnp.dot(a, t1, preferred_element_type=jnp.float32) + b1_ref[...]
    t1 = jnp.maximum(t1, 0.0)
    t2 = jnp.dot(t1, w2_ref[...], preferred_element_type=jnp.float32)
    t2 = jnp.dot(a, t2, preferred_element_type=jnp.float32) + b2_ref[...]
    t2 = jnp.maximum(t2, 0.0)
    t3 = jnp.dot(t2, w3_ref[...], preferred_element_type=jnp.float32)
    out_ref[...] = jnp.dot(a, t3, preferred_element_type=jnp.float32) + b3_ref[...]


def _temporal(t_emb, t_adj, w1, b1, w2, b2, w3, b3):
    t, d = t_emb.shape
    return pl.pallas_call(
        _temporal_body,
        out_shape=jax.ShapeDtypeStruct((t, d), jnp.float32),
    )(t_emb, t_adj, w1, b1.reshape(1, -1), w2, b2.reshape(1, -1),
      w3, b3.reshape(1, -1))


CH = 80
BLK = 6

_GD = lax.GatherDimensionNumbers(
    offset_dims=(), collapsed_slice_dims=(0,), start_index_map=(0,))


def _lane_splat(v, lane):
    idx = jnp.full((DH, 1), lane, jnp.int32)
    return lax.gather(v, idx, _GD, (1,),
                      mode=lax.GatherScatterMode.PROMISE_IN_BOUNDS)


def _make_edge_kernel(n, cpt):
    info = plsc.get_sparse_core_info()
    nc, ns = info.num_cores, info.num_subcores
    rpt = n // ns
    assert rpt * ns == n and cpt % BLK == 0
    nblk = cpt // BLK
    mesh = plsc.VectorSubcoreMesh(core_axis_name="c", subcore_axis_name="s")

    slot = lambda: (
        pltpu.VMEM((CH, DW), jnp.float32),
        pltpu.VMEM((CH, DH), jnp.float32),
        pltpu.VMEM((CH,), jnp.int32),
        pltpu.SemaphoreType.DMA,
        pltpu.SemaphoreType.DMA,
    )

    @functools.partial(
        pl.kernel,
        mesh=mesh,
        compiler_params=pltpu.CompilerParams(use_tc_tiling_on_sc=False),
        out_type=jax.ShapeDtypeStruct((nc, n, DW), jnp.float32),
        scratch_types=(
            pltpu.VMEM_SHARED((n, DW), jnp.float32),
            pltpu.VMEM((BLK * CH,), jnp.int32),
            pltpu.VMEM((BLK * CH,), jnp.int32),
        ) + slot() + slot() + slot(),
    )
    def edge_kernel(hs_hbm, sd_hbm, src_hbm, dst_hbm, zmsg_hbm, msg_out,
                    acc, srcb, dstb, *slots):
        c = lax.axis_index("c")
        s = lax.axis_index("s")
        tid = s * nc + c
        sl = [slots[:5], slots[5:10], slots[10:]]

        pltpu.sync_copy(zmsg_hbm, acc.at[pl.ds(s * rpt, rpt)])
        plsc.subcore_barrier()

        def issue_gather(b, k):
            hsv, sdv, dstsv, gsem, _ = sl[b]
            off = k * CH
            for jj in range(CH // DH):
                dstsv[pl.ds(jj * DH, DH)] = dstb[pl.ds(off + jj * DH, DH)]
            pltpu.async_copy(hs_hbm.at[srcb.at[pl.ds(off, CH)]], hsv, gsem)
            pltpu.async_copy(sd_hbm.at[dstsv], sdv, gsem)

        def wait_gather(b):
            hsv, sdv, dstsv, gsem, _ = sl[b]
            pltpu.make_async_copy(hs_hbm.at[srcb.at[pl.ds(0, CH)]],
                                  hsv, gsem).wait()
            pltpu.make_async_copy(sd_hbm.at[dstsv], sdv, gsem).wait()

        def issue_scatter(b):
            hsv, _, dstsv, _, ssem = sl[b]
            pltpu.async_copy(hsv, acc.at[dstsv], ssem, add=True)

        def wait_scatter(b):
            hsv, _, dstsv, _, ssem = sl[b]
            pltpu.make_async_copy(hsv, acc.at[dstsv], ssem).wait()

        def compute(b):
            hsv, sdv, _, _, _ = sl[b]

            @plsc.parallel_loop(0, CH, 1, unroll=4)
            def _(j):
                sc = hsv[j, pl.ds(D, DH)] + sdv[j]
                ew = jnp.exp(jnp.maximum(sc, NEG_SLOPE * sc))
                hsv[j, pl.ds(D, DH)] = ew
                for hh in range(H):
                    sp = _lane_splat(ew, hh)
                    hsv[j, pl.ds(hh * DH, DH)] = hsv[j, pl.ds(hh * DH, DH)] * sp

        def process(b):
            wait_gather(b)
            compute(b)
            issue_scatter(b)

        def blk_body(g, carry):
            base = (tid * cpt + g * BLK) * CH
            pltpu.sync_copy(src_hbm.at[pl.ds(base, BLK * CH)], srcb)
            pltpu.sync_copy(dst_hbm.at[pl.ds(base, BLK * CH)], dstb)
            for k in range(BLK):
                s_cur = k % 3
                s_prc = (k - 1) % 3
                if k >= 3:
                    wait_scatter(s_cur)
                else:
                    pl.when(g > 0)(lambda b=s_cur: wait_scatter(b))
                issue_gather(s_cur, k)
                if k >= 1:
                    process(s_prc)
                else:
                    pl.when(g > 0)(lambda b=s_prc: process(b))
            return carry

        lax.fori_loop(0, nblk, blk_body, 0)

        process((cpt - 1) % 3)
        for b in range(3):
            wait_scatter(b)

        plsc.subcore_barrier()
        pltpu.sync_copy(acc.at[pl.ds(s * rpt, rpt)],
                        msg_out.at[c, pl.ds(s * rpt, rpt)])

    return edge_kernel


def _score_mat(a):
    h, dh = a.shape
    d = h * dh
    rows = jnp.arange(d) // dh
    cols = jnp.arange(DH)
    return jnp.where(rows[:, None] == cols[None, :],
                     a.reshape(-1)[:, None], 0.0).astype(jnp.float32)


def kernel(sp_x, edge_index, t_emb, t_adj, Wg0, a_src0, a_dst0,
           Wg1, a_src1, a_dst1, W1, b1, W2, b2, W3, b3):
    n, d = sp_x.shape
    e = edge_index.shape[1]
    src = edge_index[0]
    dst = edge_index[1]

    w144_0 = jnp.concatenate([Wg0, Wg0 @ _score_mat(a_src0)], axis=1)
    wd_0 = Wg0 @ _score_mat(a_dst0)
    w144_1 = jnp.concatenate([Wg1, Wg1 @ _score_mat(a_src1)], axis=1)
    wd_1 = Wg1 @ _score_mat(a_dst1)
    r = jnp.arange(DW)
    col_head = jnp.arange(D) // DH
    p2 = ((r[:, None] - D) == col_head[None, :]).astype(jnp.float32)

    info = plsc.get_sparse_core_info()
    nc, ns = info.num_cores, info.num_subcores
    nw = nc * ns
    np2 = ((n + 8 * ns - 1) // (8 * ns)) * (8 * ns)
    rpt = np2 // ns
    zmsg = jnp.zeros((rpt, DW), jnp.float32)

    cpt = -(-e // (CH * nw))
    cpt += (-cpt) % BLK
    e2 = cpt * nw * CH
    pad_idx = (n + jnp.arange(e2 - e, dtype=jnp.int32) % (np2 - n)).astype(jnp.int32)
    src = jnp.concatenate([src, pad_idx])
    dst = jnp.concatenate([dst, pad_idx])

    edge_k = _make_edge_kernel(np2, cpt)

    bn = np2 // 16
    x0 = jnp.pad(sp_x, ((0, np2 - n), (0, 0)))
    hs1, sd1 = _dense1(x0, w144_0, wd_0, bn)
    msg1 = edge_k(hs1, sd1, src, dst, zmsg)
    hs2, sd2 = _dense2(msg1, p2, w144_1, wd_1, bn)
    msg2 = edge_k(hs2, sd2, src, dst, zmsg)
    sp = _final(msg2, p2, bn)[:n]

    tp = _temporal(t_emb, t_adj, W1, b1, W2, b2, W3, b3)
    return (sp, tp)
